# Initial kernel scaffold; baseline (speedup 1.0000x reference)
import jax
import jax.numpy as jnp
from jax.experimental import pallas as pl


def kernel(input, qweight, scales, zeros):
    raise NotImplementedError("write your pallas kernel here")


# speedup vs baseline: 2.8278x; 2.8278x over previous
import functools

import jax
import jax.numpy as jnp
from jax import lax
from jax.experimental import pallas as pl
from jax.experimental.pallas import tpu as pltpu
from jax.experimental.pallas import tpu_sc as plsc

NUM_EMBEDDINGS = 100000
EMBEDDING_DIM = 64
WORDS = EMBEDDING_DIM // 4

_info = plsc.get_sparse_core_info()
NC, NS, L = _info.num_cores, _info.num_subcores, _info.num_lanes
NW = NC * NS

CHUNK = 128
GROUPS = CHUNK // L


def _vbcast(vec, lane):
    idx = jnp.full((L,), lane, jnp.int32)
    return lax.gather(
        vec, idx[:, None],
        lax.GatherDimensionNumbers(offset_dims=(), collapsed_slice_dims=(0,),
                                   start_index_map=(0,)),
        (1,), mode=lax.GatherScatterMode.PROMISE_IN_BOUNDS)


def _vshuffle(vec, idx):
    return lax.gather(
        vec, idx[:, None],
        lax.GatherDimensionNumbers(offset_dims=(), collapsed_slice_dims=(0,),
                                   start_index_map=(0,)),
        (1,), mode=lax.GatherScatterMode.PROMISE_IN_BOUNDS)


def _dequant_body(idx_hbm, qtab_hbm, s_hbm, z_hbm, out_hbm,
                  idx_a, idx_b, q_a, q_b, s_a, s_b, z_a, z_b, out_a, out_b,
                  gsem_a, gsem_b, osem_a, osem_b, total_rows):
    wid = lax.axis_index("s") * NC + lax.axis_index("c")
    per_w = total_rows // NW
    nchunk = per_w // CHUNK
    iota16 = lax.iota(jnp.int32, L)
    shamt = (iota16 & 3) * 8
    sel = [(iota16 >> 2) + 4 * t for t in range(4)]

    bufs = ((idx_a, q_a, s_a, z_a, out_a, gsem_a, osem_a),
            (idx_b, q_b, s_b, z_b, out_b, gsem_b, osem_b))

    def start(c, buf):
        idx_v, q_v, s_v, z_v, _, gsem, _ = buf
        base = wid * per_w + c * CHUNK
        pltpu.sync_copy(idx_hbm.at[pl.ds(base, CHUNK)], idx_v)
        pltpu.async_copy(qtab_hbm.at[idx_v], q_v, gsem)
        pltpu.async_copy(s_hbm.at[idx_v], s_v, gsem)
        pltpu.async_copy(z_hbm.at[idx_v], z_v, gsem)

    def compute(buf):
        _, q_v, s_v, z_v, out_v, _, _ = buf

        def group(g, carry):
            s16 = s_v[pl.ds(g * L, L)]
            z16 = z_v[pl.ds(g * L, L)]
            for r in range(L):
                i = g * L + r
                w = q_v[i, :]
                sb = _vbcast(s16, r)
                zb = _vbcast(z16, r)
                for t in range(4):
                    b = lax.shift_right_logical(_vshuffle(w, sel[t]),
                                                shamt) & 255
                    out_v[i, pl.ds(L * t, L)] = (b.astype(jnp.float32)
                                                 - zb) * sb
            return carry

        lax.fori_loop(0, GROUPS, group, 0)

    def finish(c, buf, has_prev_store):
        idx_v, q_v, s_v, z_v, out_v, gsem, osem = buf
        base = wid * per_w + c * CHUNK
        pltpu.make_async_copy(qtab_hbm.at[idx_v], q_v, gsem).wait()
        pltpu.make_async_copy(s_hbm.at[idx_v], s_v, gsem).wait()
        pltpu.make_async_copy(z_hbm.at[idx_v], z_v, gsem).wait()

        @pl.when(has_prev_store)
        def _():
            pltpu.make_async_copy(out_v, out_hbm.at[pl.ds(base, CHUNK)],
                                  osem).wait()

        compute(buf)
        pltpu.async_copy(out_v, out_hbm.at[pl.ds(base, CHUNK)], osem)

    start(0, bufs[0])

    def step(t, carry):
        start(2 * t + 1, bufs[1])
        finish(2 * t, bufs[0], t > 0)

        @pl.when(t < (nchunk // 2) - 1)
        def _():
            start(2 * t + 2, bufs[0])

        finish(2 * t + 1, bufs[1], t > 0)
        return carry

    lax.fori_loop(0, nchunk // 2, step, 0)

    last_a = wid * per_w + (nchunk - 2) * CHUNK
    last_b = wid * per_w + (nchunk - 1) * CHUNK
    pltpu.make_async_copy(bufs[0][4], out_hbm.at[pl.ds(last_a, CHUNK)],
                          bufs[0][6]).wait()
    pltpu.make_async_copy(bufs[1][4], out_hbm.at[pl.ds(last_b, CHUNK)],
                          bufs[1][6]).wait()


@functools.partial(jax.jit, static_argnames=("total_rows",))
def _run(idx_flat, qtab, scales, zeros, total_rows):
    mesh = plsc.VectorSubcoreMesh(core_axis_name="c", subcore_axis_name="s")
    body = functools.partial(_dequant_body, total_rows=total_rows)
    return pl.kernel(
        body,
        out_type=jax.ShapeDtypeStruct((total_rows, EMBEDDING_DIM),
                                      jnp.float32),
        mesh=mesh,
        compiler_params=pltpu.CompilerParams(use_tc_tiling_on_sc=False),
        scratch_types=[
            pltpu.VMEM((CHUNK,), jnp.int32),
            pltpu.VMEM((CHUNK,), jnp.int32),
            pltpu.VMEM((CHUNK, WORDS), jnp.int32),
            pltpu.VMEM((CHUNK, WORDS), jnp.int32),
            pltpu.VMEM((CHUNK,), jnp.float32),
            pltpu.VMEM((CHUNK,), jnp.float32),
            pltpu.VMEM((CHUNK,), jnp.float32),
            pltpu.VMEM((CHUNK,), jnp.float32),
            pltpu.VMEM((CHUNK, EMBEDDING_DIM), jnp.float32),
            pltpu.VMEM((CHUNK, EMBEDDING_DIM), jnp.float32),
            pltpu.SemaphoreType.DMA,
            pltpu.SemaphoreType.DMA,
            pltpu.SemaphoreType.DMA,
            pltpu.SemaphoreType.DMA,
        ],
    )(idx_flat, qtab, scales, zeros)


def kernel(input, qweight, scales, zeros):
    batch, hist = input.shape
    total_rows = batch * hist
    idx_flat = input.reshape(total_rows).astype(jnp.int32)
    qtab = lax.bitcast_convert_type(
        qweight.reshape(NUM_EMBEDDINGS, WORDS, 4), jnp.int32)
    out = _run(idx_flat, qtab, scales, zeros, total_rows)
    return out.reshape(batch, hist, EMBEDDING_DIM)
